# baseline (speedup 1.0000x reference)
import functools

import jax
import jax.numpy as jnp
import numpy as np
from jax.experimental import pallas as pl
from jax.experimental.pallas import tpu as pltpu
from jax.experimental.pallas import tpu_sc as plsc

NEG_INF = float("-inf")


def _topk_body(nchunk, chunk, nq, hidden_ref, wqi_ref, wqo_ref, keys_ref,
               q_ref, idx_ref, qt_s, m1_s, m2_s, i1_s, i2_s):
    c = pl.program_id(0)

    @pl.when(c == 0)
    def _init():
        qin = jax.lax.dot_general(hidden_ref[...], wqi_ref[...],
                                  (((1,), (1,)), ((), ())))
        q = jax.lax.dot_general(qin, wqo_ref[...],
                                (((1,), (1,)), ((), ())))
        q_ref[...] = q
        qt_s[...] = qin[:nq].T
        m1_s[...] = jnp.full((1, nq), NEG_INF, jnp.float32)
        m2_s[...] = jnp.full((1, nq), NEG_INF, jnp.float32)
        i1_s[...] = jnp.zeros((1, nq), jnp.int32)
        i2_s[...] = jnp.zeros((1, nq), jnp.int32)

    sub = 1000
    kp = jax.lax.dot_general(keys_ref[...], wqo_ref[...],
                             (((1,), (0,)), ((), ())))
    sims = [jnp.dot(kp[o:o + sub], qt_s[...],
                    preferred_element_type=jnp.float32)
            for o in range(0, chunk, sub)]

    for simt, base in [(sims[i], c * chunk + i * sub)
                       for i in range(chunk // sub)]:
        m1 = jnp.max(simt, axis=0, keepdims=True)
        eq1 = simt == m1
        sim2 = jnp.where(eq1, NEG_INF, simt)
        m2 = jnp.max(sim2, axis=0, keepdims=True)
        eq2 = simt == m2

        iota = jax.lax.broadcasted_iota(jnp.int32, (2, sub), 1)
        ia = (iota // 256).astype(jnp.bfloat16)
        ib = (iota % 256).astype(jnp.bfloat16)
        iab = jnp.concatenate([ia[0:1], ib[1:2]], axis=0)
        oh1 = eq1.astype(jnp.bfloat16)
        oh2 = eq2.astype(jnp.bfloat16)
        r1 = jax.lax.dot_general(iab, oh1, (((1,), (0,)), ((), ())),
                                 preferred_element_type=jnp.float32)
        r2 = jax.lax.dot_general(iab, oh2, (((1,), (0,)), ((), ())),
                                 preferred_element_type=jnp.float32)
        gi1 = (r1[0:1] * 256.0 + r1[1:2]).astype(jnp.int32) + base
        gi2 = (r2[0:1] * 256.0 + r2[1:2]).astype(jnp.int32) + base

        a1, a2, ia1, ia2 = m1_s[...], m2_s[...], i1_s[...], i2_s[...]
        c1 = a1 >= m1
        n_m1 = jnp.where(c1, a1, m1)
        n_i1 = jnp.where(c1, ia1, gi1)
        cx = jnp.where(c1, a2, a1)
        ix = jnp.where(c1, ia2, ia1)
        cy = jnp.where(c1, m1, m2)
        iy = jnp.where(c1, gi1, gi2)
        c2 = cx >= cy
        m1_s[...] = n_m1
        i1_s[...] = n_i1
        m2_s[...] = jnp.where(c2, cx, cy)
        i2_s[...] = jnp.where(c2, ix, iy)

    @pl.when(c == nchunk - 1)
    def _fin():
        idx_ref[...] = jnp.concatenate([i1_s[...], i2_s[...]], axis=0)


def _topk_call(hidden, index_keys, wq_in, wq_out):
    s, h = hidden.shape
    n, d = index_keys.shape
    chunk = 5000
    assert n % chunk == 0
    nchunk = n // chunk
    nq = (s + 1) // 2
    r = wq_in.shape[0]
    q, idx = pl.pallas_call(
        functools.partial(_topk_body, nchunk, chunk, nq),
        grid=(nchunk,),
        in_specs=[
            pl.BlockSpec((s, h), lambda c: (0, 0)),
            pl.BlockSpec((r, h), lambda c: (0, 0)),
            pl.BlockSpec((d, r), lambda c: (0, 0)),
            pl.BlockSpec((chunk, d), lambda c: (c, 0)),
        ],
        out_specs=[
            pl.BlockSpec((s, d), lambda c: (0, 0)),
            pl.BlockSpec((2, nq), lambda c: (0, 0)),
        ],
        out_shape=[
            jax.ShapeDtypeStruct((s, d), jnp.float32),
            jax.ShapeDtypeStruct((2, nq), jnp.int32),
        ],
        scratch_shapes=[
            pltpu.VMEM((r, nq), jnp.float32),
            pltpu.VMEM((1, nq), jnp.float32),
            pltpu.VMEM((1, nq), jnp.float32),
            pltpu.VMEM((1, nq), jnp.int32),
            pltpu.VMEM((1, nq), jnp.int32),
        ],
        compiler_params=pltpu.CompilerParams(
            dimension_semantics=("arbitrary",)),
    )(hidden, wq_in, wq_out, index_keys)
    return q, idx


def _gather_call(index_keys, index_values, flat_idx):
    n_idx = flat_idx.shape[1]
    d = index_keys.shape[1]
    window = 128
    mesh = plsc.VectorSubcoreMesh(core_axis_name="core",
                                  subcore_axis_name="subcore")

    @pl.kernel(
        out_type=(jax.ShapeDtypeStruct((n_idx, d), jnp.float32),
                  jax.ShapeDtypeStruct((n_idx, d), jnp.float32)),
        mesh=mesh,
    )
    def gather_kernel(keys_hbm, vals_hbm, i_hbm, ok_hbm, ov_hbm):
        def body(i_vmem, ok_vmem, ov_vmem):
            pltpu.sync_copy(keys_hbm.at[i_vmem.at[0]], ok_vmem)
            pltpu.sync_copy(vals_hbm.at[i_vmem.at[0]], ov_vmem)

        pltpu.emit_pipeline(
            body,
            grid=(n_idx // window,),
            in_specs=[pl.BlockSpec((1, window), index_map=lambda i: (0, i))],
            out_specs=[
                pl.BlockSpec((window, d), index_map=lambda i: (i, 0)),
                pl.BlockSpec((window, d), index_map=lambda i: (i, 0)),
            ],
            core_axis_name="subcore",
            dimension_semantics=(pltpu.PARALLEL,),
        )(i_hbm, ok_hbm, ov_hbm)

    return gather_kernel(index_keys, index_values, flat_idx)


def _layer_body(hid_ref, wl_ref, o_ref):
    o_ref[...] = jax.lax.dot_general(hid_ref[...], wl_ref[...],
                                     (((1,), (1,)), ((), ())))


def _layer_call(hidden, w_layer):
    s, h = hidden.shape
    bq = 512
    return pl.pallas_call(
        _layer_body,
        grid=(s // bq,),
        in_specs=[
            pl.BlockSpec((bq, h), lambda b: (b, 0)),
            pl.BlockSpec((h, h), lambda b: (0, 0)),
        ],
        out_specs=pl.BlockSpec((bq, h), lambda b: (b, 0)),
        out_shape=jax.ShapeDtypeStruct((s, h), jnp.float32),
        compiler_params=pltpu.CompilerParams(
            dimension_semantics=("parallel",)),
    )(hidden, w_layer)


def _attn_body(bq, scale, q_ref, kg_ref, vg_ref, lay_ref,
               wvi_ref, wvo_ref, o_ref):
    b = pl.program_id(0)
    sk = kg_ref.shape[0]
    scores = jax.lax.dot_general(q_ref[...], kg_ref[...],
                                 (((1,), (1,)), ((), ()))) * scale
    row = b * bq + jax.lax.broadcasted_iota(jnp.int32, (bq, 1), 0)
    col = jax.lax.broadcasted_iota(jnp.int32, (1, sk), 1)
    scores = jnp.where(col <= row, scores, NEG_INF)
    m = jnp.max(scores, axis=1, keepdims=True)
    e = jnp.exp(scores - m)
    p = e / jnp.sum(e, axis=1, keepdims=True)
    mome = jnp.dot(p, vg_ref[...], preferred_element_type=jnp.float32)
    pv = jax.lax.dot_general(mome, wvi_ref[...], (((1,), (1,)), ((), ())))
    proj = jax.lax.dot_general(pv, wvo_ref[...], (((1,), (1,)), ((), ())))
    o_ref[...] = lay_ref[...] + proj


def _attn_call(q, key_g, val_g, lay, wv_in, wv_out):
    s, d = q.shape
    h = lay.shape[1]
    sk = key_g.shape[0]
    r = wv_in.shape[0]
    bq = 512
    scale = 1.0 / np.sqrt(d)
    return pl.pallas_call(
        functools.partial(_attn_body, bq, scale),
        grid=(s // bq,),
        in_specs=[
            pl.BlockSpec((bq, d), lambda b: (b, 0)),
            pl.BlockSpec((sk, d), lambda b: (0, 0)),
            pl.BlockSpec((sk, d), lambda b: (0, 0)),
            pl.BlockSpec((bq, h), lambda b: (b, 0)),
            pl.BlockSpec((r, d), lambda b: (0, 0)),
            pl.BlockSpec((h, r), lambda b: (0, 0)),
        ],
        out_specs=pl.BlockSpec((bq, h), lambda b: (b, 0)),
        out_shape=jax.ShapeDtypeStruct((s, h), jnp.float32),
        compiler_params=pltpu.CompilerParams(
            dimension_semantics=("parallel",)),
    )(q, key_g, val_g, lay, wv_in, wv_out)


def kernel(hidden_states, index_keys, index_values, W_layer,
           Wq_in, Wq_out, Wv_in, Wv_out, index_k):
    b, s, h = hidden_states.shape
    hidden = hidden_states.reshape(s, h)
    q, idx = _topk_call(hidden, index_keys, Wq_in, Wq_out)
    nq = idx.shape[1]
    flat_idx = idx.T.reshape(1, 2 * nq)
    key_g, val_g = _gather_call(index_keys, index_values, flat_idx)
    lay = _layer_call(hidden, W_layer)
    out = _attn_call(q, key_g, val_g, lay, Wv_in, Wv_out)
    return out.reshape(b, s, h)


# speedup vs baseline: 1.1256x; 1.1256x over previous
import functools

import jax
import jax.numpy as jnp
import numpy as np
from jax.experimental import pallas as pl
from jax.experimental.pallas import tpu as pltpu
from jax.experimental.pallas import tpu_sc as plsc

NEG_INF = float("-inf")


def _topk_body(nchunk, chunk, nq, hidden_ref, wqi_ref, wqo_ref, keys_ref,
               q_ref, idx_ref, qt_s, m1_s, m2_s, i1_s, i2_s):
    c = pl.program_id(0)

    @pl.when(c == 0)
    def _init():
        qin = jax.lax.dot_general(hidden_ref[...], wqi_ref[...],
                                  (((1,), (1,)), ((), ())))
        q = jax.lax.dot_general(qin, wqo_ref[...],
                                (((1,), (1,)), ((), ())))
        q_ref[...] = q
        qt_s[...] = q[:nq].T
        m1_s[...] = jnp.full((1, nq), NEG_INF, jnp.float32)
        m2_s[...] = jnp.full((1, nq), NEG_INF, jnp.float32)
        i1_s[...] = jnp.zeros((1, nq), jnp.int32)
        i2_s[...] = jnp.zeros((1, nq), jnp.int32)

    sub = 1000
    sims = [jnp.dot(keys_ref[o:o + sub], qt_s[...],
                    preferred_element_type=jnp.float32)
            for o in range(0, chunk, sub)]

    for simt, base in [(sims[i], c * chunk + i * sub)
                       for i in range(chunk // sub)]:
        m1 = jnp.max(simt, axis=0, keepdims=True)
        eq1 = simt == m1
        sim2 = jnp.where(eq1, NEG_INF, simt)
        m2 = jnp.max(sim2, axis=0, keepdims=True)
        eq2 = simt == m2

        iota = jax.lax.broadcasted_iota(jnp.int32, (2, sub), 1)
        ia = (iota // 256).astype(jnp.bfloat16)
        ib = (iota % 256).astype(jnp.bfloat16)
        iab = jnp.concatenate([ia[0:1], ib[1:2]], axis=0)
        oh1 = eq1.astype(jnp.bfloat16)
        oh2 = eq2.astype(jnp.bfloat16)
        r1 = jax.lax.dot_general(iab, oh1, (((1,), (0,)), ((), ())),
                                 preferred_element_type=jnp.float32)
        r2 = jax.lax.dot_general(iab, oh2, (((1,), (0,)), ((), ())),
                                 preferred_element_type=jnp.float32)
        gi1 = (r1[0:1] * 256.0 + r1[1:2]).astype(jnp.int32) + base
        gi2 = (r2[0:1] * 256.0 + r2[1:2]).astype(jnp.int32) + base

        a1, a2, ia1, ia2 = m1_s[...], m2_s[...], i1_s[...], i2_s[...]
        c1 = a1 >= m1
        n_m1 = jnp.where(c1, a1, m1)
        n_i1 = jnp.where(c1, ia1, gi1)
        cx = jnp.where(c1, a2, a1)
        ix = jnp.where(c1, ia2, ia1)
        cy = jnp.where(c1, m1, m2)
        iy = jnp.where(c1, gi1, gi2)
        c2 = cx >= cy
        m1_s[...] = n_m1
        i1_s[...] = n_i1
        m2_s[...] = jnp.where(c2, cx, cy)
        i2_s[...] = jnp.where(c2, ix, iy)

    @pl.when(c == nchunk - 1)
    def _fin():
        idx_ref[...] = jnp.concatenate([i1_s[...], i2_s[...]], axis=0)


def _topk_call(hidden, index_keys, wq_in, wq_out):
    s, h = hidden.shape
    n, d = index_keys.shape
    chunk = 5000
    assert n % chunk == 0
    nchunk = n // chunk
    nq = (s + 1) // 2
    r = wq_in.shape[0]
    q, idx = pl.pallas_call(
        functools.partial(_topk_body, nchunk, chunk, nq),
        grid=(nchunk,),
        in_specs=[
            pl.BlockSpec((s, h), lambda c: (0, 0)),
            pl.BlockSpec((r, h), lambda c: (0, 0)),
            pl.BlockSpec((d, r), lambda c: (0, 0)),
            pl.BlockSpec((chunk, d), lambda c: (c, 0)),
        ],
        out_specs=[
            pl.BlockSpec((s, d), lambda c: (0, 0)),
            pl.BlockSpec((2, nq), lambda c: (0, 0)),
        ],
        out_shape=[
            jax.ShapeDtypeStruct((s, d), jnp.float32),
            jax.ShapeDtypeStruct((2, nq), jnp.int32),
        ],
        scratch_shapes=[
            pltpu.VMEM((d, nq), jnp.float32),
            pltpu.VMEM((1, nq), jnp.float32),
            pltpu.VMEM((1, nq), jnp.float32),
            pltpu.VMEM((1, nq), jnp.int32),
            pltpu.VMEM((1, nq), jnp.int32),
        ],
        compiler_params=pltpu.CompilerParams(
            dimension_semantics=("arbitrary",)),
    )(hidden, wq_in, wq_out, index_keys)
    return q, idx


def _gather_call(index_keys, index_values, flat_idx):
    n_idx = flat_idx.shape[1]
    d = index_keys.shape[1]
    window = 128
    mesh = plsc.VectorSubcoreMesh(core_axis_name="core",
                                  subcore_axis_name="subcore")

    @pl.kernel(
        out_type=(jax.ShapeDtypeStruct((n_idx, d), jnp.float32),
                  jax.ShapeDtypeStruct((n_idx, d), jnp.float32)),
        mesh=mesh,
    )
    def gather_kernel(keys_hbm, vals_hbm, i_hbm, ok_hbm, ov_hbm):
        def body(i_vmem, ok_vmem, ov_vmem):
            pltpu.sync_copy(keys_hbm.at[i_vmem.at[0]], ok_vmem)
            pltpu.sync_copy(vals_hbm.at[i_vmem.at[0]], ov_vmem)

        pltpu.emit_pipeline(
            body,
            grid=(n_idx // window,),
            in_specs=[pl.BlockSpec((1, window), index_map=lambda i: (0, i))],
            out_specs=[
                pl.BlockSpec((window, d), index_map=lambda i: (i, 0)),
                pl.BlockSpec((window, d), index_map=lambda i: (i, 0)),
            ],
            core_axis_name="subcore",
            dimension_semantics=(pltpu.PARALLEL,),
        )(i_hbm, ok_hbm, ov_hbm)

    return gather_kernel(index_keys, index_values, flat_idx)


def _layer_body(hid_ref, wl_ref, o_ref):
    o_ref[...] = jax.lax.dot_general(hid_ref[...], wl_ref[...],
                                     (((1,), (1,)), ((), ())))


def _layer_call(hidden, w_layer):
    s, h = hidden.shape
    bq = 512
    return pl.pallas_call(
        _layer_body,
        grid=(s // bq,),
        in_specs=[
            pl.BlockSpec((bq, h), lambda b: (b, 0)),
            pl.BlockSpec((h, h), lambda b: (0, 0)),
        ],
        out_specs=pl.BlockSpec((bq, h), lambda b: (b, 0)),
        out_shape=jax.ShapeDtypeStruct((s, h), jnp.float32),
        compiler_params=pltpu.CompilerParams(
            dimension_semantics=("parallel",)),
    )(hidden, w_layer)


def _attn_body(bq, scale, q_ref, kg_ref, vg_ref, lay_ref,
               wvi_ref, wvo_ref, o_ref):
    b = pl.program_id(0)
    sk = kg_ref.shape[0]
    scores = jax.lax.dot_general(q_ref[...], kg_ref[...],
                                 (((1,), (1,)), ((), ()))) * scale
    row = b * bq + jax.lax.broadcasted_iota(jnp.int32, (bq, 1), 0)
    col = jax.lax.broadcasted_iota(jnp.int32, (1, sk), 1)
    scores = jnp.where(col <= row, scores, NEG_INF)
    m = jnp.max(scores, axis=1, keepdims=True)
    e = jnp.exp(scores - m)
    p = e / jnp.sum(e, axis=1, keepdims=True)
    mome = jnp.dot(p, vg_ref[...], preferred_element_type=jnp.float32)
    pv = jax.lax.dot_general(mome, wvi_ref[...], (((1,), (1,)), ((), ())))
    proj = jax.lax.dot_general(pv, wvo_ref[...], (((1,), (1,)), ((), ())))
    o_ref[...] = lay_ref[...] + proj


def _attn_call(q, key_g, val_g, lay, wv_in, wv_out):
    s, d = q.shape
    h = lay.shape[1]
    sk = key_g.shape[0]
    r = wv_in.shape[0]
    bq = 512
    scale = 1.0 / np.sqrt(d)
    return pl.pallas_call(
        functools.partial(_attn_body, bq, scale),
        grid=(s // bq,),
        in_specs=[
            pl.BlockSpec((bq, d), lambda b: (b, 0)),
            pl.BlockSpec((sk, d), lambda b: (0, 0)),
            pl.BlockSpec((sk, d), lambda b: (0, 0)),
            pl.BlockSpec((bq, h), lambda b: (b, 0)),
            pl.BlockSpec((r, d), lambda b: (0, 0)),
            pl.BlockSpec((h, r), lambda b: (0, 0)),
        ],
        out_specs=pl.BlockSpec((bq, h), lambda b: (b, 0)),
        out_shape=jax.ShapeDtypeStruct((s, h), jnp.float32),
        compiler_params=pltpu.CompilerParams(
            dimension_semantics=("parallel",)),
    )(q, key_g, val_g, lay, wv_in, wv_out)


def kernel(hidden_states, index_keys, index_values, W_layer,
           Wq_in, Wq_out, Wv_in, Wv_out, index_k):
    b, s, h = hidden_states.shape
    hidden = hidden_states.reshape(s, h)
    q, idx = _topk_call(hidden, index_keys, Wq_in, Wq_out)
    nq = idx.shape[1]
    flat_idx = idx.T.reshape(1, 2 * nq)
    key_g, val_g = _gather_call(index_keys, index_values, flat_idx)
    lay = _layer_call(hidden, W_layer)
    out = _attn_call(q, key_g, val_g, lay, Wv_in, Wv_out)
    return out.reshape(b, s, h)
